# baseline (speedup 1.0000x reference)
import functools

import jax
import jax.numpy as jnp
from jax.experimental import pallas as pl
from jax.experimental.pallas import tpu as pltpu

_IN = 3
_HID = 64
_OUT = 2
_SEG = 8


def _mlp_kernel(xq_ref, w1b_ref, b1b_ref, w2b_ref, b2b_ref, o_ref):
    h = jnp.dot(w1b_ref[...], xq_ref[...],
                preferred_element_type=jnp.float32)
    hb = h.astype(jnp.bfloat16) + b1b_ref[...]
    hb = jnp.maximum(hb, jnp.bfloat16(0.0))
    y = jnp.dot(w2b_ref[...], hb, preferred_element_type=jnp.float32)
    o_ref[...] = y + b2b_ref[...]


@functools.partial(jax.jit, static_argnames=("w",))
def _forward(x, w1, b1, w2, b2, *, w=16384):
    n = x.shape[0]
    n8 = n // _SEG
    assert n % (_SEG * w) == 0
    grid = (n8 // w,)

    xq = x.T.reshape(_IN * _SEG, n8)

    eye = jnp.eye(_SEG, dtype=jnp.float32)
    w1big = (eye[:, None, None, :] * w1[None, :, :, None]
             ).reshape(_SEG * _HID, _IN * _SEG).astype(jnp.bfloat16)
    w2big = (eye[:, None, :, None] * w2[None, :, None, :]
             ).reshape(_SEG * _OUT, _SEG * _HID).astype(jnp.bfloat16)
    b1big = jnp.tile(b1, _SEG).reshape(_SEG * _HID, 1).astype(jnp.bfloat16)
    b2big = jnp.tile(b2, _SEG).reshape(_SEG * _OUT, 1)

    yq = pl.pallas_call(
        _mlp_kernel,
        out_shape=jax.ShapeDtypeStruct((_SEG * _OUT, n8), jnp.float32),
        grid_spec=pl.GridSpec(
            grid=grid,
            in_specs=[
                pl.BlockSpec((_IN * _SEG, w), lambda i: (0, i)),
                pl.BlockSpec((_SEG * _HID, _IN * _SEG), lambda i: (0, 0)),
                pl.BlockSpec((_SEG * _HID, 1), lambda i: (0, 0)),
                pl.BlockSpec((_SEG * _OUT, _SEG * _HID), lambda i: (0, 0)),
                pl.BlockSpec((_SEG * _OUT, 1), lambda i: (0, 0)),
            ],
            out_specs=pl.BlockSpec((_SEG * _OUT, w), lambda i: (0, i)),
        ),
        compiler_params=pltpu.CompilerParams(
            dimension_semantics=("parallel",),
        ),
    )(xq, w1big, b1big, w2big, b2big)

    return yq.reshape(_SEG, _OUT, n8).transpose(1, 0, 2).reshape(_OUT, n).T


def kernel(x, w1, b1, w2, b2):
    return _forward(x, w1, b1, w2, b2, w=16384)


# speedup vs baseline: 2.0126x; 2.0126x over previous
import functools

import jax
import jax.numpy as jnp
from jax.experimental import pallas as pl
from jax.experimental.pallas import tpu as pltpu

_IN = 3
_HID = 64
_OUT = 2


def _mlp_kernel(xt_ref, w1_ref, b1_ref, w2_ref, b2_ref, o_ref):
    xt = xt_ref[...]
    h = jnp.dot(w1_ref[...], xt, preferred_element_type=jnp.float32)
    h = jnp.maximum(h + b1_ref[...], 0.0)
    y = jnp.dot(w2_ref[...], h, preferred_element_type=jnp.float32) + b2_ref[...]
    o_ref[...] = y.astype(o_ref.dtype)


@functools.partial(jax.jit, static_argnames=("tn",))
def _forward(x, w1, b1, w2, b2, *, tn=262144):
    n = x.shape[0]
    n_128 = max(128, ((n + 127) // 128) * 128)
    tile = min(tn, n_128)
    n_pad = ((n_128 + tile - 1) // tile) * tile
    grid = (n_pad // tile,)

    xt = jnp.pad(x.T, ((0, 0), (0, n_pad - n)))
    b1c = b1.reshape(_HID, 1)
    b2c = b2.reshape(_OUT, 1)

    out_t = pl.pallas_call(
        _mlp_kernel,
        out_shape=jax.ShapeDtypeStruct((_OUT, n_pad), jnp.float32),
        grid_spec=pl.GridSpec(
            grid=grid,
            in_specs=[
                pl.BlockSpec((_IN, tile), lambda i: (0, i)),
                pl.BlockSpec((_HID, _IN), lambda i: (0, 0)),
                pl.BlockSpec((_HID, 1), lambda i: (0, 0)),
                pl.BlockSpec((_OUT, _HID), lambda i: (0, 0)),
                pl.BlockSpec((_OUT, 1), lambda i: (0, 0)),
            ],
            out_specs=pl.BlockSpec((_OUT, tile), lambda i: (0, i)),
        ),
        compiler_params=pltpu.CompilerParams(
            dimension_semantics=("parallel",),
        ),
    )(xt, w1, b1c, w2, b2c)

    return out_t[:, :n].T


def kernel(x, w1, b1, w2, b2):
    return _forward(x, w1, b1, w2, b2, tn=262144)


# speedup vs baseline: 2.0137x; 1.0006x over previous
import functools

import jax
import jax.numpy as jnp
from jax.experimental import pallas as pl
from jax.experimental.pallas import tpu as pltpu

_IN = 3
_HID = 64
_OUT = 2


def _mlp_kernel(xt_ref, w1_ref, b1_ref, w2_ref, b2_ref, o_ref):
    xt = xt_ref[...]
    h = jnp.dot(w1_ref[...], xt, preferred_element_type=jnp.float32)
    h = jnp.maximum(h + b1_ref[...], 0.0)
    y = jnp.dot(w2_ref[...], h, preferred_element_type=jnp.float32) + b2_ref[...]
    o_ref[...] = y.astype(o_ref.dtype)


@functools.partial(jax.jit, static_argnames=("tn",))
def _forward(x, w1, b1, w2, b2, *, tn=262144):
    n = x.shape[0]
    n_128 = max(128, ((n + 127) // 128) * 128)
    tile = min(tn, n_128)
    n_pad = ((n_128 + tile - 1) // tile) * tile
    grid = (n_pad // tile,)

    xt = x.T if n_pad == n else jnp.pad(x.T, ((0, 0), (0, n_pad - n)))
    b1c = b1.reshape(_HID, 1)
    b2c = b2.reshape(_OUT, 1)

    out_t = pl.pallas_call(
        _mlp_kernel,
        out_shape=jax.ShapeDtypeStruct((_OUT, n_pad), jnp.float32),
        grid_spec=pl.GridSpec(
            grid=grid,
            in_specs=[
                pl.BlockSpec((_IN, tile), lambda i: (0, i)),
                pl.BlockSpec((_HID, _IN), lambda i: (0, 0)),
                pl.BlockSpec((_HID, 1), lambda i: (0, 0)),
                pl.BlockSpec((_OUT, _HID), lambda i: (0, 0)),
                pl.BlockSpec((_OUT, 1), lambda i: (0, 0)),
            ],
            out_specs=pl.BlockSpec((_OUT, tile), lambda i: (0, i)),
        ),
        compiler_params=pltpu.CompilerParams(
            dimension_semantics=("parallel",),
        ),
    )(xt, w1, b1c, w2, b2c)

    return out_t[:, :n].T


def kernel(x, w1, b1, w2, b2):
    return _forward(x, w1, b1, w2, b2, tn=262144)
